# baseline (speedup 1.0000x reference)
import functools

import jax
import jax.numpy as jnp
from jax import lax
from jax.experimental import pallas as pl
from jax.experimental.pallas import tpu as pltpu
from jax.experimental.pallas import tpu_sc as plsc

F32 = jnp.float32


def _dot(a, b):
    return jnp.dot(a, b, preferred_element_type=F32)


_SC_CHUNK = 128
_SC_NCHUNK = 40
_SC_PER_W = _SC_CHUNK * _SC_NCHUNK
_SC_NW = 32
_SC_EPAD = _SC_PER_W * _SC_NW
_SC_TOTCH = _SC_NCHUNK * _SC_NW


_SC_NBUF = 4
_SC_NGRP = _SC_NCHUNK // _SC_NBUF


def _sc_gather(table, idx_pad):
    d = table.shape[1]
    dt = table.dtype
    mesh = plsc.VectorSubcoreMesh(core_axis_name="c", subcore_axis_name="s")

    @functools.partial(
        pl.kernel,
        out_type=jax.ShapeDtypeStruct((_SC_TOTCH, _SC_CHUNK, d), dt),
        mesh=mesh,
        scratch_types=(
            [pltpu.VMEM((_SC_PER_W,), jnp.int32)]
            + [pltpu.VMEM((_SC_CHUNK, d), dt) for _ in range(_SC_NBUF)]
            + [pltpu.SemaphoreType.DMA for _ in range(2 * _SC_NBUF)]
        ),
        compiler_params=pltpu.CompilerParams(use_tc_tiling_on_sc=False),
    )
    def k(table_hbm, idx_hbm, out_hbm, idx_v, *bufsem):
        bufs = bufsem[:_SC_NBUF]
        gsems = bufsem[_SC_NBUF:2 * _SC_NBUF]
        wsems = bufsem[2 * _SC_NBUF:]
        wid = lax.axis_index("s") * 2 + lax.axis_index("c")
        base = wid * _SC_PER_W
        cbase = wid * _SC_NCHUNK
        pltpu.sync_copy(idx_hbm.at[pl.ds(base, _SC_PER_W)], idx_v)

        def gather(c, b):
            return pltpu.async_copy(
                table_hbm.at[idx_v.at[pl.ds(c * _SC_CHUNK, _SC_CHUNK)]],
                bufs[b], gsems[b])

        for b in range(_SC_NBUF):
            gather(b, b)

        def grp_body(gi, _):
            c0 = gi * _SC_NBUF
            for b in range(_SC_NBUF):
                pltpu.make_async_copy(
                    table_hbm.at[idx_v.at[pl.ds(0, _SC_CHUNK)]],
                    bufs[b], gsems[b]).wait()
                pltpu.async_copy(bufs[b], out_hbm.at[cbase + c0 + b], wsems[b])
            for b in range(_SC_NBUF):
                @pl.when(gi < _SC_NGRP - 1)
                def _(b=b):
                    pltpu.make_async_copy(
                        bufs[b], out_hbm.at[cbase + c0 + b], wsems[b]).wait()
                    gather(c0 + _SC_NBUF + b, b)
            return 0

        lax.fori_loop(0, _SC_NGRP, grp_body, 0, unroll=False)
        for b in range(_SC_NBUF):
            pltpu.make_async_copy(
                bufs[b], out_hbm.at[cbase + (_SC_NGRP - 1) * _SC_NBUF + b],
                wsems[b]).wait()

    return k(table, idx_pad)


def _prologue_body(af_ref, sf_ref, wat, ba, wst, bs, want, wsnt, waet, wset, be1,
                   atom_ref, state_ref, pas_ref, base_ref):
    atom = _dot(af_ref[...], wat[...]) + ba[...]
    state = _dot(sf_ref[...], wst[...]) + bs[...]
    atom_ref[...] = atom
    state_ref[...] = state
    pas_ref[...] = _dot(atom, want[...]) + _dot(state, wsnt[...])
    base_ref[...] = _dot(atom, waet[...]) + _dot(state, wset[...]) + be1[...]


def _prologue(atom_fea, state_fea, wat, ba, wst, bs, want, wsnt, waet, wset, be1):
    n = atom_fea.shape[0]
    return pl.pallas_call(
        _prologue_body,
        out_shape=(
            jax.ShapeDtypeStruct((n, 32), F32),
            jax.ShapeDtypeStruct((n, 32), F32),
            jax.ShapeDtypeStruct((n, 64), F32),
            jax.ShapeDtypeStruct((n, 64), F32),
        ),
    )(atom_fea, state_fea, wat, ba, wst, bs, want, wsnt, waet, wset, be1)


_NB = 400
_R4 = _NB * 4
_NGRID = 25
_E4 = _R4 * _NGRID


def _pass1_body(first, *args):
    if first:
        (raw_ref, gpa_ref, gpb_ref, base_ref, wnt_ref,
         bn_ref, w1q_ref, w2d_ref, b2p_ref,
         epa_ref, epb_ref, nbrp_out_ref, ssum_ref, ssq_ref) = args
        raw = raw_ref[...]
        nf = raw.shape[-1]
        embs = [_dot(raw[:, 4 * q:4 * q + 4, :].reshape(_R4, nf), wnt_ref[...])
                + bn_ref[...] for q in range(4)]
        nbrp = jnp.concatenate(embs, axis=1)
        nbrp_out_ref[...] = nbrp
    else:
        (nbr_ref, gpa_ref, gpb_ref, base_ref,
         w1q_ref, w2d_ref, b2p_ref,
         epa_ref, epb_ref, ssum_ref, ssq_ref) = args
        nbrp = nbr_ref[...]
    znbr4 = _dot(nbrp, w1q_ref[...])
    base = base_ref[...]
    br = jnp.broadcast_to(base[:, None, :], (_NB, 4, 64)).reshape(_R4, 64)
    basep = jnp.concatenate([br, br], axis=1)
    zpa = jnp.concatenate([znbr4[:, 0:64], znbr4[:, 128:192]], axis=1) \
        + gpa_ref[...].astype(F32) + basep
    zpb = jnp.concatenate([znbr4[:, 64:128], znbr4[:, 192:256]], axis=1) \
        + gpb_ref[...].astype(F32) + basep
    epa = _dot(jnp.tanh(zpa), w2d_ref[...]) + b2p_ref[...]
    epb = _dot(jnp.tanh(zpb), w2d_ref[...]) + b2p_ref[...]
    epa_ref[...] = epa
    epb_ref[...] = epb
    part = (jnp.sum(epa.reshape(_R4 // 8, 8, 128), axis=0)
            + jnp.sum(epb.reshape(_R4 // 8, 8, 128), axis=0))
    part2 = (jnp.sum((epa * epa).reshape(_R4 // 8, 8, 128), axis=0)
             + jnp.sum((epb * epb).reshape(_R4 // 8, 8, 128), axis=0))

    @pl.when(pl.program_id(0) == 0)
    def _():
        ssum_ref[...] = part
        ssq_ref[...] = part2

    @pl.when(pl.program_id(0) != 0)
    def _():
        ssum_ref[...] += part
        ssq_ref[...] += part2


def _pass1(first, nbr_in, gp, base, wnt, bn, w1q, w2d, b2p):
    full = lambda s: pl.BlockSpec(s, lambda i: (0, 0))
    out_shape = [jax.ShapeDtypeStruct((_E4, 128), F32)] * 2
    out_specs = [pl.BlockSpec((_R4, 128), lambda i: (i, 0))] * 2
    if first:
        out_shape.append(jax.ShapeDtypeStruct((_E4, 128), F32))
        out_specs.append(pl.BlockSpec((_R4, 128), lambda i: (i, 0)))
    out_shape += [jax.ShapeDtypeStruct((8, 128), F32)] * 2
    out_specs += [pl.BlockSpec((8, 128), lambda i: (0, 0))] * 2
    gp_specs = [pl.BlockSpec((_R4, 128), lambda i: (i, 0)),
                pl.BlockSpec((_R4, 128), lambda i: (i + _NGRID, 0))]
    base_spec = pl.BlockSpec((_NB, 64), lambda i: (i, 0))
    w_specs = [full((128, 256)), full((128, 128)), full((1, 128))]
    if first:
        nf = nbr_in.shape[-1]
        raw_spec = pl.BlockSpec((_NB, 16, nf), lambda i: (i, 0, 0))
        in_specs = ([raw_spec] + gp_specs + [base_spec]
                    + [full((nf, 32)), full((1, 32))] + w_specs)
        ins = (nbr_in, gp, gp, base, wnt, bn, w1q, w2d, b2p)
    else:
        in_specs = ([pl.BlockSpec((_R4, 128), lambda i: (i, 0))]
                    + gp_specs + [base_spec] + w_specs)
        ins = (nbr_in, gp, gp, base, w1q, w2d, b2p)
    return pl.pallas_call(
        functools.partial(_pass1_body, first),
        grid=(_NGRID,),
        in_specs=in_specs,
        out_specs=tuple(out_specs),
        out_shape=tuple(out_shape),
    )(*ins)


def _pass2_body(epa_ref, epb_ref, nbrp_ref, scale_ref, shift_ref,
                eop_ref, em_ref):
    eha = epa_ref[...] * scale_ref[...] + shift_ref[...]
    ehb = epb_ref[...] * scale_ref[...] + shift_ref[...]
    pa0 = eha[:, 0:32] * eha[:, 32:64]
    pa2 = eha[:, 64:96] * eha[:, 96:128]
    pb1 = ehb[:, 0:32] * ehb[:, 32:64]
    pb3 = ehb[:, 64:96] * ehb[:, 96:128]
    eop = jnp.concatenate([pa0, pb1, pa2, pb3], axis=1) + nbrp_ref[...]
    eop_ref[...] = eop
    s = jnp.sum(eop.reshape(_NB, 4, 128), axis=1)
    em_ref[...] = (s[:, 0:32] + s[:, 32:64] + s[:, 64:96] + s[:, 96:128]) \
        * (1.0 / 16.0)


def _pass2(epa, epb, nbrp, scalep, shiftp):
    n = _NB * _NGRID
    full = lambda s: pl.BlockSpec(s, lambda i: (0, 0))
    return pl.pallas_call(
        _pass2_body,
        grid=(_NGRID,),
        in_specs=[
            pl.BlockSpec((_R4, 128), lambda i: (i, 0)),
            pl.BlockSpec((_R4, 128), lambda i: (i, 0)),
            pl.BlockSpec((_R4, 128), lambda i: (i, 0)),
            full((1, 128)),
            full((1, 128)),
        ],
        out_specs=(pl.BlockSpec((_R4, 128), lambda i: (i, 0)),
                   pl.BlockSpec((_NB, 32), lambda i: (i, 0))),
        out_shape=(jax.ShapeDtypeStruct((_E4, 128), F32),
                   jax.ShapeDtypeStruct((n, 32), F32)),
    )(epa, epb, nbrp, scalep, shiftp)


def _bn_cols(x, g, b):
    n = x.shape[0]
    m = jnp.sum(x, axis=0, keepdims=True) / n
    var = jnp.sum(x * x, axis=0, keepdims=True) / n - m * m
    return (x - m) * (g / jnp.sqrt(var + 1e-5)) + b


def _node_body(em_ref, atom_ref, state_ref, v1t, bv1, v2t, bv2,
               g2, bt2, ut, bu, g3, bt3, want, wsnt, waet, wset, be1n,
               atom_o, state_o, pas_o, base_o):
    x = jnp.concatenate([em_ref[...], atom_ref[...]], axis=1)
    h = jnp.tanh(_dot(x, v1t[...]) + bv1[...])
    v = jnp.tanh(_dot(h, v2t[...]) + bv2[...])
    vn = _bn_cols(v, g2[...], bt2[...])
    atom_new = vn[:, :32] * vn[:, 32:] + atom_ref[...]
    su = _dot(state_ref[...], ut[...]) + bu[...]
    state_new = _bn_cols(su, g3[...], bt3[...]) + state_ref[...]
    atom_o[...] = atom_new
    state_o[...] = state_new
    pas_o[...] = _dot(atom_new, want[...]) + _dot(state_new, wsnt[...])
    base_o[...] = _dot(atom_new, waet[...]) + _dot(state_new, wset[...]) + be1n[...]


def _node(em, atom, state, v1t, bv1, v2t, bv2, g2, bt2, ut, bu, g3, bt3,
          want, wsnt, waet, wset, be1n):
    n = atom.shape[0]
    return pl.pallas_call(
        _node_body,
        out_shape=(
            jax.ShapeDtypeStruct((n, 32), F32),
            jax.ShapeDtypeStruct((n, 32), F32),
            jax.ShapeDtypeStruct((n, 64), F32),
            jax.ShapeDtypeStruct((n, 64), F32),
        ),
    )(em, atom, state, v1t, bv1, v2t, bv2, g2, bt2, ut, bu, g3, bt3,
      want, wsnt, waet, wset, be1n)


def _node_last_body(em_ref, atom_ref, state_ref, v1t, bv1, v2t,
                    bv2, g2, bt2, ut, bu, g3, bt3, pooled_o):
    em = em_ref[...]
    x = jnp.concatenate([em, atom_ref[...]], axis=1)
    h = jnp.tanh(_dot(x, v1t[...]) + bv1[...])
    v = jnp.tanh(_dot(h, v2t[...]) + bv2[...])
    vn = _bn_cols(v, g2[...], bt2[...])
    atom_new = vn[:, :32] * vn[:, 32:] + atom_ref[...]
    su = _dot(state_ref[...], ut[...]) + bu[...]
    state_new = _bn_cols(su, g3[...], bt3[...]) + state_ref[...]
    crys = jnp.concatenate([em * 16.0, atom_new, state_new], axis=1)
    n = crys.shape[0]
    pooled_o[...] = jnp.mean(crys.reshape(n // 50, 50, 96), axis=1)


def _node_last(em, atom, state, v1t, bv1, v2t, bv2, g2, bt2, ut, bu, g3, bt3):
    n = atom.shape[0]
    return pl.pallas_call(
        _node_last_body,
        out_shape=jax.ShapeDtypeStruct((n // 50, 96), F32),
    )(em, atom, state, v1t, bv1, v2t, bv2, g2, bt2, ut, bu, g3, bt3)


def _conv1_body(x_ref, w_ref, b_ref, o_ref):
    x = x_ref[...]
    bsz = x.shape[0]
    outs = []
    for o in range(3):
        acc = jnp.full((bsz, 27, 27), b_ref[o], dtype=F32)
        for dh in range(2):
            for dw in range(2):
                acc = acc + w_ref[o, 0, dh, dw] * x[:, dh:dh + 27, dw:dw + 27]
        m = jnp.maximum(
            jnp.maximum(acc[:, 0:26, 0:26], acc[:, 0:26, 1:27]),
            jnp.maximum(acc[:, 1:27, 0:26], acc[:, 1:27, 1:27]),
        )
        outs.append(jnp.maximum(m, 0.0))
    o_ref[...] = jnp.stack(outs, axis=0)


def _conv2_body(x_ref, w_ref, b_ref, o_ref):
    x = x_ref[...]
    bsz = x.shape[1]
    outs = []
    for o in range(6):
        acc = jnp.full((bsz, 12, 12), b_ref[o], dtype=F32)
        for ci in range(3):
            for dh in range(2):
                for dw in range(2):
                    acc = acc + w_ref[o, ci, dh, dw] * x[ci, :, dh:dh + 12, dw:dw + 12]
        m = jnp.maximum(
            jnp.maximum(acc[:, 0:11, 0:11], acc[:, 0:11, 1:12]),
            jnp.maximum(acc[:, 1:12, 0:11], acc[:, 1:12, 1:12]),
        )
        outs.append(jnp.maximum(m, 0.0))
    o_ref[...] = jnp.stack(outs, axis=0)


def _head_body(pooled_ref, sflat_ref, wf3t, bf3, wfc1t, bfc1, wfot, bfo, o_ref):
    s12 = jnp.tanh(_dot(sflat_ref[...], wf3t[...]) + bf3[...])
    x = jnp.concatenate([pooled_ref[...], s12], axis=1)
    h = jnp.tanh(_dot(x, wfc1t[...]) + bfc1[...])
    o_ref[...] = _dot(h, wfot[...]) + bfo[...]


def _split_e1(p):
    w = p["e1"]["w"]
    return (w[:, 64:96].T, w[:, 96:128].T, w[:, 128:160].T,
            w[:, 0:32].T, w[:, 32:64].T, p["e1"]["b"].reshape(1, 64))


def _blockdiag(wt, copies):
    i, o = wt.shape
    z = jnp.zeros((copies, i, copies, o), F32)
    ii = jnp.arange(copies)
    z = z.at[ii, :, ii, :].set(wt)
    return z.reshape(copies * i, copies * o)


def kernel(atom_fea, nbr_fea, nbr_fea_idx, state_fea, surface_fea,
           crystal_atom_idx, params):
    p = params
    blocks = p["blocks"]
    n, m = nbr_fea_idx.shape
    e_rows = n * m

    e1s = [_split_e1(bp) for bp in blocks]

    atom, state, pas, base = _prologue(
        atom_fea, state_fea,
        p["emb_a"]["w"].T, p["emb_a"]["b"].reshape(1, 32),
        p["emb_s"]["w"].T, p["emb_s"]["b"].reshape(1, 32),
        e1s[0][1], e1s[0][2], e1s[0][3], e1s[0][4], e1s[0][5])

    idx_a = jnp.stack([nbr_fea_idx[:, 0:4], nbr_fea_idx[:, 8:12]],
                      axis=2).reshape(e_rows // 2)
    idx_b = jnp.stack([nbr_fea_idx[:, 4:8], nbr_fea_idx[:, 12:16]],
                      axis=2).reshape(e_rows // 2)
    idx_pad = jnp.concatenate(
        [idx_a, idx_b, jnp.zeros((_SC_EPAD - e_rows,), jnp.int32)])

    nbr_cur = nbr_fea
    for k in range(3):
        bp = blocks[k]
        wnbr_t, _, _, _, _, be1 = e1s[k]
        w1q = _blockdiag(wnbr_t, 4)
        w2d = _blockdiag(bp["e2"]["w"].T, 2)
        b2 = bp["e2"]["b"].reshape(1, 64)
        b2p = jnp.concatenate([b2, b2], axis=1)
        g3d = _sc_gather(pas.astype(jnp.bfloat16), idx_pad)
        gp = g3d.reshape(_SC_EPAD * 64 // 128, 128)
        if k == 0:
            epa, epb, nbr_cur, ssum, ssq = _pass1(
                True, nbr_cur, gp, base,
                p["emb_n"]["w"].T, p["emb_n"]["b"].reshape(1, 32),
                w1q, w2d, b2p)
        else:
            epa, epb, ssum, ssq = _pass1(
                False, nbr_cur, gp, base, None, None, w1q, w2d, b2p)
        s128 = jnp.sum(ssum, axis=0)
        q128 = jnp.sum(ssq, axis=0)
        s = (s128[:64] + s128[64:]).reshape(1, 64)
        q = (q128[:64] + q128[64:]).reshape(1, 64)
        mean = s / e_rows
        var = q / e_rows - mean * mean
        scale = bp["g1"].reshape(1, 64) / jnp.sqrt(var + 1e-5)
        shift = bp["bt1"].reshape(1, 64) - mean * scale
        scalep = jnp.concatenate([scale, scale], axis=1)
        shiftp = jnp.concatenate([shift, shift], axis=1)
        nbr_cur, em = _pass2(epa, epb, nbr_cur, scalep, shiftp)

        nargs = (bp["v1"]["w"].T, bp["v1"]["b"].reshape(1, 64),
                 bp["v2"]["w"].T, bp["v2"]["b"].reshape(1, 64),
                 bp["g2"].reshape(1, 64), bp["bt2"].reshape(1, 64),
                 bp["u"]["w"].T, bp["u"]["b"].reshape(1, 32),
                 bp["g3"].reshape(1, 32), bp["bt3"].reshape(1, 32))
        if k < 2:
            nx = e1s[k + 1]
            atom, state, pas, base = _node(
                em, atom, state, *nargs, nx[1], nx[2], nx[3], nx[4], nx[5])
        else:
            pooled = _node_last(em, atom, state, *nargs)

    bsz = surface_fea.shape[0]
    xs = surface_fea[:, 0, :, :]
    smem = pl.BlockSpec(memory_space=pltpu.SMEM)
    c1 = pl.pallas_call(
        _conv1_body,
        in_specs=[pl.BlockSpec(), smem, smem],
        out_shape=jax.ShapeDtypeStruct((3, bsz, 26, 26), F32),
    )(xs, p["c1w"], p["c1b"])
    c1d = c1[:, :, ::2, ::2]
    c2 = pl.pallas_call(
        _conv2_body,
        in_specs=[pl.BlockSpec(), smem, smem],
        out_shape=jax.ShapeDtypeStruct((6, bsz, 11, 11), F32),
    )(c1d, p["c2w"], p["c2b"])
    c2d = c2[:, :, ::2, ::2]
    sflat = jnp.transpose(c2d, (1, 0, 2, 3)).reshape(bsz, 216)

    out = pl.pallas_call(
        _head_body,
        out_shape=jax.ShapeDtypeStruct((bsz, 1), F32),
    )(pooled, sflat,
      p["f3"]["w"].T, p["f3"]["b"].reshape(1, 12),
      p["fc1"]["w"].T, p["fc1"]["b"].reshape(1, 128),
      p["fo"]["w"].T, p["fo"]["b"].reshape(1, 1))
    return out


# speedup vs baseline: 1.0286x; 1.0286x over previous
import functools

import jax
import jax.numpy as jnp
from jax import lax
from jax.experimental import pallas as pl
from jax.experimental.pallas import tpu as pltpu
from jax.experimental.pallas import tpu_sc as plsc

F32 = jnp.float32


def _dot(a, b):
    return jnp.dot(a, b, preferred_element_type=F32)


_SC_CHUNK = 128
_SC_NCHUNK = 40
_SC_PER_W = _SC_CHUNK * _SC_NCHUNK
_SC_NW = 32
_SC_EPAD = _SC_PER_W * _SC_NW
_SC_TOTCH = _SC_NCHUNK * _SC_NW


_SC_NBUF = 4
_SC_NGRP = _SC_NCHUNK // _SC_NBUF


def _sc_gather(table, idx_pad):
    d = table.shape[1]
    dt = table.dtype
    mesh = plsc.VectorSubcoreMesh(core_axis_name="c", subcore_axis_name="s")

    @functools.partial(
        pl.kernel,
        out_type=jax.ShapeDtypeStruct((_SC_TOTCH, _SC_CHUNK, d), dt),
        mesh=mesh,
        scratch_types=(
            [pltpu.VMEM((_SC_PER_W,), jnp.int32)]
            + [pltpu.VMEM((_SC_CHUNK, d), dt) for _ in range(_SC_NBUF)]
            + [pltpu.SemaphoreType.DMA for _ in range(2 * _SC_NBUF)]
        ),
        compiler_params=pltpu.CompilerParams(use_tc_tiling_on_sc=False),
    )
    def k(table_hbm, idx_hbm, out_hbm, idx_v, *bufsem):
        bufs = bufsem[:_SC_NBUF]
        gsems = bufsem[_SC_NBUF:2 * _SC_NBUF]
        wsems = bufsem[2 * _SC_NBUF:]
        wid = lax.axis_index("s") * 2 + lax.axis_index("c")
        base = wid * _SC_PER_W
        cbase = wid * _SC_NCHUNK
        pltpu.sync_copy(idx_hbm.at[pl.ds(base, _SC_PER_W)], idx_v)

        def gather(c, b):
            return pltpu.async_copy(
                table_hbm.at[idx_v.at[pl.ds(c * _SC_CHUNK, _SC_CHUNK)]],
                bufs[b], gsems[b])

        for b in range(_SC_NBUF):
            gather(b, b)

        def grp_body(gi, _):
            c0 = gi * _SC_NBUF
            for b in range(_SC_NBUF):
                pltpu.make_async_copy(
                    table_hbm.at[idx_v.at[pl.ds(0, _SC_CHUNK)]],
                    bufs[b], gsems[b]).wait()
                pltpu.async_copy(bufs[b], out_hbm.at[cbase + c0 + b], wsems[b])
            for b in range(_SC_NBUF):
                @pl.when(gi < _SC_NGRP - 1)
                def _(b=b):
                    pltpu.make_async_copy(
                        bufs[b], out_hbm.at[cbase + c0 + b], wsems[b]).wait()
                    gather(c0 + _SC_NBUF + b, b)
            return 0

        lax.fori_loop(0, _SC_NGRP, grp_body, 0, unroll=False)
        for b in range(_SC_NBUF):
            pltpu.make_async_copy(
                bufs[b], out_hbm.at[cbase + (_SC_NGRP - 1) * _SC_NBUF + b],
                wsems[b]).wait()

    return k(table, idx_pad)


def _prologue_body(af_ref, sf_ref, wat, ba, wst, bs, want, wsnt, waet, wset, be1,
                   atom_ref, state_ref, pas_ref, base_ref):
    atom = _dot(af_ref[...], wat[...]) + ba[...]
    state = _dot(sf_ref[...], wst[...]) + bs[...]
    atom_ref[...] = atom
    state_ref[...] = state
    pas_ref[...] = (_dot(atom, want[...]) + _dot(state, wsnt[...])
                    ).astype(jnp.bfloat16)
    base_ref[...] = _dot(atom, waet[...]) + _dot(state, wset[...]) + be1[...]


def _prologue(atom_fea, state_fea, wat, ba, wst, bs, want, wsnt, waet, wset, be1):
    n = atom_fea.shape[0]
    return pl.pallas_call(
        _prologue_body,
        out_shape=(
            jax.ShapeDtypeStruct((n, 32), F32),
            jax.ShapeDtypeStruct((n, 32), F32),
            jax.ShapeDtypeStruct((n, 64), jnp.bfloat16),
            jax.ShapeDtypeStruct((n, 64), F32),
        ),
    )(atom_fea, state_fea, wat, ba, wst, bs, want, wsnt, waet, wset, be1)


_NB = 400
_R4 = _NB * 4
_NGRID = 25
_E4 = _R4 * _NGRID


def _pass1_body(first, *args):
    if first:
        (raw_ref, gpa_ref, gpb_ref, base_ref, wnt_ref,
         bn_ref, w1q_ref, w2d_ref, b2p_ref,
         epa_ref, epb_ref, nbrp_out_ref, ssum_ref, ssq_ref) = args
        raw = raw_ref[...]
        nf = raw.shape[-1]
        embs = [_dot(raw[:, 4 * q:4 * q + 4, :].reshape(_R4, nf), wnt_ref[...])
                + bn_ref[...] for q in range(4)]
        nbrp = jnp.concatenate(embs, axis=1)
        nbrp_out_ref[...] = nbrp
    else:
        (nbr_ref, gpa_ref, gpb_ref, base_ref,
         w1q_ref, w2d_ref, b2p_ref,
         epa_ref, epb_ref, ssum_ref, ssq_ref) = args
        nbrp = nbr_ref[...]
    znbr4 = _dot(nbrp, w1q_ref[...])
    base = base_ref[...]
    br = jnp.broadcast_to(base[:, None, :], (_NB, 4, 64)).reshape(_R4, 64)
    basep = jnp.concatenate([br, br], axis=1)
    zpa = jnp.concatenate([znbr4[:, 0:64], znbr4[:, 128:192]], axis=1) \
        + gpa_ref[...].astype(F32) + basep
    zpb = jnp.concatenate([znbr4[:, 64:128], znbr4[:, 192:256]], axis=1) \
        + gpb_ref[...].astype(F32) + basep
    epa = _dot(jnp.tanh(zpa), w2d_ref[...]) + b2p_ref[...]
    epb = _dot(jnp.tanh(zpb), w2d_ref[...]) + b2p_ref[...]
    epa_ref[...] = epa
    epb_ref[...] = epb
    part = (jnp.sum(epa.reshape(_R4 // 8, 8, 128), axis=0)
            + jnp.sum(epb.reshape(_R4 // 8, 8, 128), axis=0))
    part2 = (jnp.sum((epa * epa).reshape(_R4 // 8, 8, 128), axis=0)
             + jnp.sum((epb * epb).reshape(_R4 // 8, 8, 128), axis=0))

    @pl.when(pl.program_id(0) == 0)
    def _():
        ssum_ref[...] = part
        ssq_ref[...] = part2

    @pl.when(pl.program_id(0) != 0)
    def _():
        ssum_ref[...] += part
        ssq_ref[...] += part2


def _pass1(first, nbr_in, gp, base, wnt, bn, w1q, w2d, b2p):
    full = lambda s: pl.BlockSpec(s, lambda i: (0, 0))
    out_shape = [jax.ShapeDtypeStruct((_E4, 128), F32)] * 2
    out_specs = [pl.BlockSpec((_R4, 128), lambda i: (i, 0))] * 2
    if first:
        out_shape.append(jax.ShapeDtypeStruct((_E4, 128), F32))
        out_specs.append(pl.BlockSpec((_R4, 128), lambda i: (i, 0)))
    out_shape += [jax.ShapeDtypeStruct((8, 128), F32)] * 2
    out_specs += [pl.BlockSpec((8, 128), lambda i: (0, 0))] * 2
    gp_specs = [pl.BlockSpec((_R4, 128), lambda i: (i, 0)),
                pl.BlockSpec((_R4, 128), lambda i: (i + _NGRID, 0))]
    base_spec = pl.BlockSpec((_NB, 64), lambda i: (i, 0))
    w_specs = [full((128, 256)), full((128, 128)), full((1, 128))]
    if first:
        nf = nbr_in.shape[-1]
        raw_spec = pl.BlockSpec((_NB, 16, nf), lambda i: (i, 0, 0))
        in_specs = ([raw_spec] + gp_specs + [base_spec]
                    + [full((nf, 32)), full((1, 32))] + w_specs)
        ins = (nbr_in, gp, gp, base, wnt, bn, w1q, w2d, b2p)
    else:
        in_specs = ([pl.BlockSpec((_R4, 128), lambda i: (i, 0))]
                    + gp_specs + [base_spec] + w_specs)
        ins = (nbr_in, gp, gp, base, w1q, w2d, b2p)
    return pl.pallas_call(
        functools.partial(_pass1_body, first),
        grid=(_NGRID,),
        in_specs=in_specs,
        out_specs=tuple(out_specs),
        out_shape=tuple(out_shape),
    )(*ins)


def _pass2_body(epa_ref, epb_ref, nbrp_ref, scale_ref, shift_ref,
                eop_ref, em_ref):
    eha = epa_ref[...] * scale_ref[...] + shift_ref[...]
    ehb = epb_ref[...] * scale_ref[...] + shift_ref[...]
    pa0 = eha[:, 0:32] * eha[:, 32:64]
    pa2 = eha[:, 64:96] * eha[:, 96:128]
    pb1 = ehb[:, 0:32] * ehb[:, 32:64]
    pb3 = ehb[:, 64:96] * ehb[:, 96:128]
    eop = jnp.concatenate([pa0, pb1, pa2, pb3], axis=1) + nbrp_ref[...]
    eop_ref[...] = eop
    s = jnp.sum(eop.reshape(_NB, 4, 128), axis=1)
    em_ref[...] = (s[:, 0:32] + s[:, 32:64] + s[:, 64:96] + s[:, 96:128]) \
        * (1.0 / 16.0)


def _pass2(epa, epb, nbrp, scalep, shiftp):
    n = _NB * _NGRID
    full = lambda s: pl.BlockSpec(s, lambda i: (0, 0))
    return pl.pallas_call(
        _pass2_body,
        grid=(_NGRID,),
        in_specs=[
            pl.BlockSpec((_R4, 128), lambda i: (i, 0)),
            pl.BlockSpec((_R4, 128), lambda i: (i, 0)),
            pl.BlockSpec((_R4, 128), lambda i: (i, 0)),
            full((1, 128)),
            full((1, 128)),
        ],
        out_specs=(pl.BlockSpec((_R4, 128), lambda i: (i, 0)),
                   pl.BlockSpec((_NB, 32), lambda i: (i, 0))),
        out_shape=(jax.ShapeDtypeStruct((_E4, 128), F32),
                   jax.ShapeDtypeStruct((n, 32), F32)),
    )(epa, epb, nbrp, scalep, shiftp)


def _bn_cols(x, g, b):
    n = x.shape[0]
    m = jnp.sum(x, axis=0, keepdims=True) / n
    var = jnp.sum(x * x, axis=0, keepdims=True) / n - m * m
    return (x - m) * (g / jnp.sqrt(var + 1e-5)) + b


def _node_body(em_ref, atom_ref, state_ref, v1t, bv1, v2t, bv2,
               g2, bt2, ut, bu, g3, bt3, want, wsnt, waet, wset, be1n,
               atom_o, state_o, pas_o, base_o):
    x = jnp.concatenate([em_ref[...], atom_ref[...]], axis=1)
    h = jnp.tanh(_dot(x, v1t[...]) + bv1[...])
    v = jnp.tanh(_dot(h, v2t[...]) + bv2[...])
    vn = _bn_cols(v, g2[...], bt2[...])
    atom_new = vn[:, :32] * vn[:, 32:] + atom_ref[...]
    su = _dot(state_ref[...], ut[...]) + bu[...]
    state_new = _bn_cols(su, g3[...], bt3[...]) + state_ref[...]
    atom_o[...] = atom_new
    state_o[...] = state_new
    pas_o[...] = (_dot(atom_new, want[...]) + _dot(state_new, wsnt[...])
                  ).astype(jnp.bfloat16)
    base_o[...] = _dot(atom_new, waet[...]) + _dot(state_new, wset[...]) + be1n[...]


def _node(em, atom, state, v1t, bv1, v2t, bv2, g2, bt2, ut, bu, g3, bt3,
          want, wsnt, waet, wset, be1n):
    n = atom.shape[0]
    return pl.pallas_call(
        _node_body,
        out_shape=(
            jax.ShapeDtypeStruct((n, 32), F32),
            jax.ShapeDtypeStruct((n, 32), F32),
            jax.ShapeDtypeStruct((n, 64), jnp.bfloat16),
            jax.ShapeDtypeStruct((n, 64), F32),
        ),
    )(em, atom, state, v1t, bv1, v2t, bv2, g2, bt2, ut, bu, g3, bt3,
      want, wsnt, waet, wset, be1n)


def _node_last_body(em_ref, atom_ref, state_ref, v1t, bv1, v2t,
                    bv2, g2, bt2, ut, bu, g3, bt3, pooled_o):
    em = em_ref[...]
    x = jnp.concatenate([em, atom_ref[...]], axis=1)
    h = jnp.tanh(_dot(x, v1t[...]) + bv1[...])
    v = jnp.tanh(_dot(h, v2t[...]) + bv2[...])
    vn = _bn_cols(v, g2[...], bt2[...])
    atom_new = vn[:, :32] * vn[:, 32:] + atom_ref[...]
    su = _dot(state_ref[...], ut[...]) + bu[...]
    state_new = _bn_cols(su, g3[...], bt3[...]) + state_ref[...]
    crys = jnp.concatenate([em * 16.0, atom_new, state_new], axis=1)
    n = crys.shape[0]
    pooled_o[...] = jnp.mean(crys.reshape(n // 50, 50, 96), axis=1)


def _node_last(em, atom, state, v1t, bv1, v2t, bv2, g2, bt2, ut, bu, g3, bt3):
    n = atom.shape[0]
    return pl.pallas_call(
        _node_last_body,
        out_shape=jax.ShapeDtypeStruct((n // 50, 96), F32),
    )(em, atom, state, v1t, bv1, v2t, bv2, g2, bt2, ut, bu, g3, bt3)


def _conv1_body(x_ref, w_ref, b_ref, o_ref):
    x = x_ref[...]
    bsz = x.shape[0]
    outs = []
    for o in range(3):
        acc = jnp.full((bsz, 27, 27), b_ref[o], dtype=F32)
        for dh in range(2):
            for dw in range(2):
                acc = acc + w_ref[o, 0, dh, dw] * x[:, dh:dh + 27, dw:dw + 27]
        m = jnp.maximum(
            jnp.maximum(acc[:, 0:26, 0:26], acc[:, 0:26, 1:27]),
            jnp.maximum(acc[:, 1:27, 0:26], acc[:, 1:27, 1:27]),
        )
        outs.append(jnp.maximum(m, 0.0))
    o_ref[...] = jnp.stack(outs, axis=0)


def _conv2_body(x_ref, w_ref, b_ref, o_ref):
    x = x_ref[...]
    bsz = x.shape[1]
    outs = []
    for o in range(6):
        acc = jnp.full((bsz, 12, 12), b_ref[o], dtype=F32)
        for ci in range(3):
            for dh in range(2):
                for dw in range(2):
                    acc = acc + w_ref[o, ci, dh, dw] * x[ci, :, dh:dh + 12, dw:dw + 12]
        m = jnp.maximum(
            jnp.maximum(acc[:, 0:11, 0:11], acc[:, 0:11, 1:12]),
            jnp.maximum(acc[:, 1:12, 0:11], acc[:, 1:12, 1:12]),
        )
        outs.append(jnp.maximum(m, 0.0))
    o_ref[...] = jnp.stack(outs, axis=0)


def _head_body(pooled_ref, sflat_ref, wf3t, bf3, wfc1t, bfc1, wfot, bfo, o_ref):
    s12 = jnp.tanh(_dot(sflat_ref[...], wf3t[...]) + bf3[...])
    x = jnp.concatenate([pooled_ref[...], s12], axis=1)
    h = jnp.tanh(_dot(x, wfc1t[...]) + bfc1[...])
    o_ref[...] = _dot(h, wfot[...]) + bfo[...]


def _split_e1(p):
    w = p["e1"]["w"]
    return (w[:, 64:96].T, w[:, 96:128].T, w[:, 128:160].T,
            w[:, 0:32].T, w[:, 32:64].T, p["e1"]["b"].reshape(1, 64))


def _blockdiag(wt, copies):
    i, o = wt.shape
    z = jnp.zeros((copies, i, copies, o), F32)
    ii = jnp.arange(copies)
    z = z.at[ii, :, ii, :].set(wt)
    return z.reshape(copies * i, copies * o)


def kernel(atom_fea, nbr_fea, nbr_fea_idx, state_fea, surface_fea,
           crystal_atom_idx, params):
    p = params
    blocks = p["blocks"]
    n, m = nbr_fea_idx.shape
    e_rows = n * m

    e1s = [_split_e1(bp) for bp in blocks]

    atom, state, pas, base = _prologue(
        atom_fea, state_fea,
        p["emb_a"]["w"].T, p["emb_a"]["b"].reshape(1, 32),
        p["emb_s"]["w"].T, p["emb_s"]["b"].reshape(1, 32),
        e1s[0][1], e1s[0][2], e1s[0][3], e1s[0][4], e1s[0][5])

    idx_flat = nbr_fea_idx.reshape(e_rows)
    pp = jnp.arange(_SC_EPAD, dtype=jnp.int32)
    half = jnp.where(pp < e_rows // 2, 0, 1)
    q = pp - half * (e_rows // 2)
    n_ = q >> 3
    jr = (q >> 1) & 3
    h = q & 1
    src = n_ * 16 + half * 4 + jr + h * 8
    src = jnp.where(pp < e_rows, src, 0)
    idx_pad = jnp.take(idx_flat, src)

    nbr_cur = nbr_fea
    for k in range(3):
        bp = blocks[k]
        wnbr_t, _, _, _, _, be1 = e1s[k]
        w1q = _blockdiag(wnbr_t, 4)
        w2d = _blockdiag(bp["e2"]["w"].T, 2)
        b2 = bp["e2"]["b"].reshape(1, 64)
        b2p = jnp.concatenate([b2, b2], axis=1)
        g3d = _sc_gather(pas, idx_pad)
        gp = g3d.reshape(_SC_EPAD * 64 // 128, 128)
        if k == 0:
            epa, epb, nbr_cur, ssum, ssq = _pass1(
                True, nbr_cur, gp, base,
                p["emb_n"]["w"].T, p["emb_n"]["b"].reshape(1, 32),
                w1q, w2d, b2p)
        else:
            epa, epb, ssum, ssq = _pass1(
                False, nbr_cur, gp, base, None, None, w1q, w2d, b2p)
        s128 = jnp.sum(ssum, axis=0)
        q128 = jnp.sum(ssq, axis=0)
        s = (s128[:64] + s128[64:]).reshape(1, 64)
        q = (q128[:64] + q128[64:]).reshape(1, 64)
        mean = s / e_rows
        var = q / e_rows - mean * mean
        scale = bp["g1"].reshape(1, 64) / jnp.sqrt(var + 1e-5)
        shift = bp["bt1"].reshape(1, 64) - mean * scale
        scalep = jnp.concatenate([scale, scale], axis=1)
        shiftp = jnp.concatenate([shift, shift], axis=1)
        nbr_cur, em = _pass2(epa, epb, nbr_cur, scalep, shiftp)

        nargs = (bp["v1"]["w"].T, bp["v1"]["b"].reshape(1, 64),
                 bp["v2"]["w"].T, bp["v2"]["b"].reshape(1, 64),
                 bp["g2"].reshape(1, 64), bp["bt2"].reshape(1, 64),
                 bp["u"]["w"].T, bp["u"]["b"].reshape(1, 32),
                 bp["g3"].reshape(1, 32), bp["bt3"].reshape(1, 32))
        if k < 2:
            nx = e1s[k + 1]
            atom, state, pas, base = _node(
                em, atom, state, *nargs, nx[1], nx[2], nx[3], nx[4], nx[5])
        else:
            pooled = _node_last(em, atom, state, *nargs)

    bsz = surface_fea.shape[0]
    xs = surface_fea[:, 0, :, :]
    smem = pl.BlockSpec(memory_space=pltpu.SMEM)
    c1 = pl.pallas_call(
        _conv1_body,
        in_specs=[pl.BlockSpec(), smem, smem],
        out_shape=jax.ShapeDtypeStruct((3, bsz, 26, 26), F32),
    )(xs, p["c1w"], p["c1b"])
    c1d = c1[:, :, ::2, ::2]
    c2 = pl.pallas_call(
        _conv2_body,
        in_specs=[pl.BlockSpec(), smem, smem],
        out_shape=jax.ShapeDtypeStruct((6, bsz, 11, 11), F32),
    )(c1d, p["c2w"], p["c2b"])
    c2d = c2[:, :, ::2, ::2]
    sflat = jnp.transpose(c2d, (1, 0, 2, 3)).reshape(bsz, 216)

    out = pl.pallas_call(
        _head_body,
        out_shape=jax.ShapeDtypeStruct((bsz, 1), F32),
    )(pooled, sflat,
      p["f3"]["w"].T, p["f3"]["b"].reshape(1, 12),
      p["fc1"]["w"].T, p["fc1"]["b"].reshape(1, 128),
      p["fo"]["w"].T, p["fo"]["b"].reshape(1, 1))
    return out


# speedup vs baseline: 1.2037x; 1.1702x over previous
import functools

import jax
import jax.numpy as jnp
from jax import lax
from jax.experimental import pallas as pl
from jax.experimental.pallas import tpu as pltpu
from jax.experimental.pallas import tpu_sc as plsc

F32 = jnp.float32


def _dot(a, b):
    return jnp.dot(a, b, preferred_element_type=F32)


_SC_CHUNK = 128
_SC_NCHUNK = 40
_SC_PER_W = _SC_CHUNK * _SC_NCHUNK
_SC_NW = 32
_SC_EPAD = _SC_PER_W * _SC_NW
_SC_TOTCH = _SC_NCHUNK * _SC_NW


_SC_NBUF = 4
_SC_NGRP = _SC_NCHUNK // _SC_NBUF


def _sc_gather(table, idx_pad):
    d = table.shape[1]
    dt = table.dtype
    mesh = plsc.VectorSubcoreMesh(core_axis_name="c", subcore_axis_name="s")

    @functools.partial(
        pl.kernel,
        out_type=jax.ShapeDtypeStruct((_SC_TOTCH, _SC_CHUNK, d), dt),
        mesh=mesh,
        scratch_types=(
            [pltpu.VMEM((_SC_PER_W,), jnp.int32)]
            + [pltpu.VMEM((_SC_CHUNK, d), dt) for _ in range(_SC_NBUF)]
            + [pltpu.SemaphoreType.DMA for _ in range(2 * _SC_NBUF)]
        ),
        compiler_params=pltpu.CompilerParams(use_tc_tiling_on_sc=False),
    )
    def k(table_hbm, idx_hbm, out_hbm, idx_v, *bufsem):
        bufs = bufsem[:_SC_NBUF]
        gsems = bufsem[_SC_NBUF:2 * _SC_NBUF]
        wsems = bufsem[2 * _SC_NBUF:]
        wid = lax.axis_index("s") * 2 + lax.axis_index("c")
        base = wid * _SC_PER_W
        cbase = wid * _SC_NCHUNK
        pltpu.sync_copy(idx_hbm.at[pl.ds(base, _SC_PER_W)], idx_v)

        def gather(c, b):
            return pltpu.async_copy(
                table_hbm.at[idx_v.at[pl.ds(c * _SC_CHUNK, _SC_CHUNK)]],
                bufs[b], gsems[b])

        for b in range(_SC_NBUF):
            gather(b, b)

        def grp_body(gi, _):
            c0 = gi * _SC_NBUF
            for b in range(_SC_NBUF):
                pltpu.make_async_copy(
                    table_hbm.at[idx_v.at[pl.ds(0, _SC_CHUNK)]],
                    bufs[b], gsems[b]).wait()
                pltpu.async_copy(bufs[b], out_hbm.at[cbase + c0 + b], wsems[b])
            for b in range(_SC_NBUF):
                @pl.when(gi < _SC_NGRP - 1)
                def _(b=b):
                    pltpu.make_async_copy(
                        bufs[b], out_hbm.at[cbase + c0 + b], wsems[b]).wait()
                    gather(c0 + _SC_NBUF + b, b)
            return 0

        lax.fori_loop(0, _SC_NGRP, grp_body, 0, unroll=False)
        for b in range(_SC_NBUF):
            pltpu.make_async_copy(
                bufs[b], out_hbm.at[cbase + (_SC_NGRP - 1) * _SC_NBUF + b],
                wsems[b]).wait()

    return k(table, idx_pad)


def _prologue_body(af_ref, sf_ref, wat, ba, wst, bs, want, wsnt, waet, wset, be1,
                   atom_ref, state_ref, pas_ref, base_ref):
    atom = _dot(af_ref[...], wat[...]) + ba[...]
    state = _dot(sf_ref[...], wst[...]) + bs[...]
    atom_ref[...] = atom
    state_ref[...] = state
    pas_ref[...] = _dot(atom, want[...]) + _dot(state, wsnt[...])
    base_ref[...] = _dot(atom, waet[...]) + _dot(state, wset[...]) + be1[...]


def _prologue(atom_fea, state_fea, wat, ba, wst, bs, want, wsnt, waet, wset, be1):
    n = atom_fea.shape[0]
    return pl.pallas_call(
        _prologue_body,
        out_shape=(
            jax.ShapeDtypeStruct((n, 32), F32),
            jax.ShapeDtypeStruct((n, 32), F32),
            jax.ShapeDtypeStruct((n, 64), F32),
            jax.ShapeDtypeStruct((n, 64), F32),
        ),
    )(atom_fea, state_fea, wat, ba, wst, bs, want, wsnt, waet, wset, be1)


_NB = 400
_R4 = _NB * 4
_NGRID = 25
_E4 = _R4 * _NGRID


def _pass1_body(first, *args):
    if first:
        (raw_ref, gpa_ref, gpb_ref, base_ref, wnt_ref,
         bn_ref, w1q_ref, w2d_ref, b2p_ref,
         epa_ref, epb_ref, nbrp_out_ref, ssum_ref, ssq_ref) = args
        raw = raw_ref[...]
        nf = raw.shape[-1]
        embs = [_dot(raw[:, 4 * q:4 * q + 4, :].reshape(_R4, nf), wnt_ref[...])
                + bn_ref[...] for q in range(4)]
        nbrp = jnp.concatenate(embs, axis=1)
        nbrp_out_ref[...] = nbrp
    else:
        (nbr_ref, gpa_ref, gpb_ref, base_ref,
         w1q_ref, w2d_ref, b2p_ref,
         epa_ref, epb_ref, ssum_ref, ssq_ref) = args
        nbrp = nbr_ref[...]
    znbr4 = _dot(nbrp, w1q_ref[...])
    base = base_ref[...]
    br = jnp.broadcast_to(base[:, None, :], (_NB, 4, 64)).reshape(_R4, 64)
    basep = jnp.concatenate([br, br], axis=1)
    zpa = jnp.concatenate([znbr4[:, 0:64], znbr4[:, 128:192]], axis=1) \
        + gpa_ref[...] + basep
    zpb = jnp.concatenate([znbr4[:, 64:128], znbr4[:, 192:256]], axis=1) \
        + gpb_ref[...] + basep
    epa = _dot(jnp.tanh(zpa), w2d_ref[...]) + b2p_ref[...]
    epb = _dot(jnp.tanh(zpb), w2d_ref[...]) + b2p_ref[...]
    epa_ref[...] = epa
    epb_ref[...] = epb
    part = (jnp.sum(epa.reshape(_R4 // 8, 8, 128), axis=0)
            + jnp.sum(epb.reshape(_R4 // 8, 8, 128), axis=0))
    part2 = (jnp.sum((epa * epa).reshape(_R4 // 8, 8, 128), axis=0)
             + jnp.sum((epb * epb).reshape(_R4 // 8, 8, 128), axis=0))

    @pl.when(pl.program_id(0) == 0)
    def _():
        ssum_ref[...] = part
        ssq_ref[...] = part2

    @pl.when(pl.program_id(0) != 0)
    def _():
        ssum_ref[...] += part
        ssq_ref[...] += part2


def _pass1(first, nbr_in, gp, base, wnt, bn, w1q, w2d, b2p):
    full = lambda s: pl.BlockSpec(s, lambda i: (0, 0))
    out_shape = [jax.ShapeDtypeStruct((_E4, 128), F32)] * 2
    out_specs = [pl.BlockSpec((_R4, 128), lambda i: (i, 0))] * 2
    if first:
        out_shape.append(jax.ShapeDtypeStruct((_E4, 128), F32))
        out_specs.append(pl.BlockSpec((_R4, 128), lambda i: (i, 0)))
    out_shape += [jax.ShapeDtypeStruct((8, 128), F32)] * 2
    out_specs += [pl.BlockSpec((8, 128), lambda i: (0, 0))] * 2
    gp_specs = [pl.BlockSpec((_R4, 128), lambda i: (i, 0)),
                pl.BlockSpec((_R4, 128), lambda i: (i + _NGRID, 0))]
    base_spec = pl.BlockSpec((_NB, 64), lambda i: (i, 0))
    w_specs = [full((128, 256)), full((128, 128)), full((1, 128))]
    if first:
        nf = nbr_in.shape[-1]
        raw_spec = pl.BlockSpec((_NB, 16, nf), lambda i: (i, 0, 0))
        in_specs = ([raw_spec] + gp_specs + [base_spec]
                    + [full((nf, 32)), full((1, 32))] + w_specs)
        ins = (nbr_in, gp, gp, base, wnt, bn, w1q, w2d, b2p)
    else:
        in_specs = ([pl.BlockSpec((_R4, 128), lambda i: (i, 0))]
                    + gp_specs + [base_spec] + w_specs)
        ins = (nbr_in, gp, gp, base, w1q, w2d, b2p)
    return pl.pallas_call(
        functools.partial(_pass1_body, first),
        grid=(_NGRID,),
        in_specs=in_specs,
        out_specs=tuple(out_specs),
        out_shape=tuple(out_shape),
    )(*ins)


def _pass2_body(epa_ref, epb_ref, nbrp_ref, scale_ref, shift_ref,
                eop_ref, em_ref):
    eha = epa_ref[...] * scale_ref[...] + shift_ref[...]
    ehb = epb_ref[...] * scale_ref[...] + shift_ref[...]
    pa0 = eha[:, 0:32] * eha[:, 32:64]
    pa2 = eha[:, 64:96] * eha[:, 96:128]
    pb1 = ehb[:, 0:32] * ehb[:, 32:64]
    pb3 = ehb[:, 64:96] * ehb[:, 96:128]
    eop = jnp.concatenate([pa0, pb1, pa2, pb3], axis=1) + nbrp_ref[...]
    eop_ref[...] = eop
    s = jnp.sum(eop.reshape(_NB, 4, 128), axis=1)
    em_ref[...] = (s[:, 0:32] + s[:, 32:64] + s[:, 64:96] + s[:, 96:128]) \
        * (1.0 / 16.0)


def _pass2(epa, epb, nbrp, scalep, shiftp):
    n = _NB * _NGRID
    full = lambda s: pl.BlockSpec(s, lambda i: (0, 0))
    return pl.pallas_call(
        _pass2_body,
        grid=(_NGRID,),
        in_specs=[
            pl.BlockSpec((_R4, 128), lambda i: (i, 0)),
            pl.BlockSpec((_R4, 128), lambda i: (i, 0)),
            pl.BlockSpec((_R4, 128), lambda i: (i, 0)),
            full((1, 128)),
            full((1, 128)),
        ],
        out_specs=(pl.BlockSpec((_R4, 128), lambda i: (i, 0)),
                   pl.BlockSpec((_NB, 32), lambda i: (i, 0))),
        out_shape=(jax.ShapeDtypeStruct((_E4, 128), F32),
                   jax.ShapeDtypeStruct((n, 32), F32)),
    )(epa, epb, nbrp, scalep, shiftp)


def _bn_cols(x, g, b):
    n = x.shape[0]
    m = jnp.sum(x, axis=0, keepdims=True) / n
    var = jnp.sum(x * x, axis=0, keepdims=True) / n - m * m
    return (x - m) * (g / jnp.sqrt(var + 1e-5)) + b


def _node_body(em_ref, atom_ref, state_ref, v1t, bv1, v2t, bv2,
               g2, bt2, ut, bu, g3, bt3, want, wsnt, waet, wset, be1n,
               atom_o, state_o, pas_o, base_o):
    x = jnp.concatenate([em_ref[...], atom_ref[...]], axis=1)
    h = jnp.tanh(_dot(x, v1t[...]) + bv1[...])
    v = jnp.tanh(_dot(h, v2t[...]) + bv2[...])
    vn = _bn_cols(v, g2[...], bt2[...])
    atom_new = vn[:, :32] * vn[:, 32:] + atom_ref[...]
    su = _dot(state_ref[...], ut[...]) + bu[...]
    state_new = _bn_cols(su, g3[...], bt3[...]) + state_ref[...]
    atom_o[...] = atom_new
    state_o[...] = state_new
    pas_o[...] = _dot(atom_new, want[...]) + _dot(state_new, wsnt[...])
    base_o[...] = _dot(atom_new, waet[...]) + _dot(state_new, wset[...]) + be1n[...]


def _node(em, atom, state, v1t, bv1, v2t, bv2, g2, bt2, ut, bu, g3, bt3,
          want, wsnt, waet, wset, be1n):
    n = atom.shape[0]
    return pl.pallas_call(
        _node_body,
        out_shape=(
            jax.ShapeDtypeStruct((n, 32), F32),
            jax.ShapeDtypeStruct((n, 32), F32),
            jax.ShapeDtypeStruct((n, 64), F32),
            jax.ShapeDtypeStruct((n, 64), F32),
        ),
    )(em, atom, state, v1t, bv1, v2t, bv2, g2, bt2, ut, bu, g3, bt3,
      want, wsnt, waet, wset, be1n)


def _node_last_body(em_ref, atom_ref, state_ref, v1t, bv1, v2t,
                    bv2, g2, bt2, ut, bu, g3, bt3, pooled_o):
    em = em_ref[...]
    x = jnp.concatenate([em, atom_ref[...]], axis=1)
    h = jnp.tanh(_dot(x, v1t[...]) + bv1[...])
    v = jnp.tanh(_dot(h, v2t[...]) + bv2[...])
    vn = _bn_cols(v, g2[...], bt2[...])
    atom_new = vn[:, :32] * vn[:, 32:] + atom_ref[...]
    su = _dot(state_ref[...], ut[...]) + bu[...]
    state_new = _bn_cols(su, g3[...], bt3[...]) + state_ref[...]
    crys = jnp.concatenate([em * 16.0, atom_new, state_new], axis=1)
    n = crys.shape[0]
    pooled_o[...] = jnp.mean(crys.reshape(n // 50, 50, 96), axis=1)


def _node_last(em, atom, state, v1t, bv1, v2t, bv2, g2, bt2, ut, bu, g3, bt3):
    n = atom.shape[0]
    return pl.pallas_call(
        _node_last_body,
        out_shape=jax.ShapeDtypeStruct((n // 50, 96), F32),
    )(em, atom, state, v1t, bv1, v2t, bv2, g2, bt2, ut, bu, g3, bt3)


def _conv1_body(x_ref, w_ref, b_ref, o_ref):
    x = x_ref[...]
    bsz = x.shape[0]
    outs = []
    for o in range(3):
        acc = jnp.full((bsz, 27, 27), b_ref[o], dtype=F32)
        for dh in range(2):
            for dw in range(2):
                acc = acc + w_ref[o, 0, dh, dw] * x[:, dh:dh + 27, dw:dw + 27]
        m = jnp.maximum(
            jnp.maximum(acc[:, 0:26, 0:26], acc[:, 0:26, 1:27]),
            jnp.maximum(acc[:, 1:27, 0:26], acc[:, 1:27, 1:27]),
        )
        outs.append(jnp.maximum(m, 0.0))
    o_ref[...] = jnp.stack(outs, axis=0)


def _conv2_body(x_ref, w_ref, b_ref, o_ref):
    x = x_ref[...]
    bsz = x.shape[1]
    outs = []
    for o in range(6):
        acc = jnp.full((bsz, 12, 12), b_ref[o], dtype=F32)
        for ci in range(3):
            for dh in range(2):
                for dw in range(2):
                    acc = acc + w_ref[o, ci, dh, dw] * x[ci, :, dh:dh + 12, dw:dw + 12]
        m = jnp.maximum(
            jnp.maximum(acc[:, 0:11, 0:11], acc[:, 0:11, 1:12]),
            jnp.maximum(acc[:, 1:12, 0:11], acc[:, 1:12, 1:12]),
        )
        outs.append(jnp.maximum(m, 0.0))
    o_ref[...] = jnp.stack(outs, axis=0)


def _head_body(pooled_ref, sflat_ref, wf3t, bf3, wfc1t, bfc1, wfot, bfo, o_ref):
    s12 = jnp.tanh(_dot(sflat_ref[...], wf3t[...]) + bf3[...])
    x = jnp.concatenate([pooled_ref[...], s12], axis=1)
    h = jnp.tanh(_dot(x, wfc1t[...]) + bfc1[...])
    o_ref[...] = _dot(h, wfot[...]) + bfo[...]


def _split_e1(p):
    w = p["e1"]["w"]
    return (w[:, 64:96].T, w[:, 96:128].T, w[:, 128:160].T,
            w[:, 0:32].T, w[:, 32:64].T, p["e1"]["b"].reshape(1, 64))


def _blockdiag(wt, copies):
    i, o = wt.shape
    z = jnp.zeros((copies, i, copies, o), F32)
    ii = jnp.arange(copies)
    z = z.at[ii, :, ii, :].set(wt)
    return z.reshape(copies * i, copies * o)


def kernel(atom_fea, nbr_fea, nbr_fea_idx, state_fea, surface_fea,
           crystal_atom_idx, params):
    p = params
    blocks = p["blocks"]
    n, m = nbr_fea_idx.shape
    e_rows = n * m

    e1s = [_split_e1(bp) for bp in blocks]

    atom, state, pas, base = _prologue(
        atom_fea, state_fea,
        p["emb_a"]["w"].T, p["emb_a"]["b"].reshape(1, 32),
        p["emb_s"]["w"].T, p["emb_s"]["b"].reshape(1, 32),
        e1s[0][1], e1s[0][2], e1s[0][3], e1s[0][4], e1s[0][5])

    idx_flat = nbr_fea_idx.reshape(e_rows)
    pp = jnp.arange(_SC_EPAD, dtype=jnp.int32)
    half = jnp.where(pp < e_rows // 2, 0, 1)
    q = pp - half * (e_rows // 2)
    n_ = q >> 3
    jr = (q >> 1) & 3
    h = q & 1
    src = n_ * 16 + half * 4 + jr + h * 8
    src = jnp.where(pp < e_rows, src, 0)
    idx_pad = jnp.take(idx_flat, src)

    nbr_cur = nbr_fea
    for k in range(3):
        bp = blocks[k]
        wnbr_t, _, _, _, _, be1 = e1s[k]
        w1q = _blockdiag(wnbr_t, 4)
        w2d = _blockdiag(bp["e2"]["w"].T, 2)
        b2 = bp["e2"]["b"].reshape(1, 64)
        b2p = jnp.concatenate([b2, b2], axis=1)
        g3d = _sc_gather(pas, idx_pad)
        gp = g3d.reshape(_SC_EPAD * 64 // 128, 128)
        if k == 0:
            epa, epb, nbr_cur, ssum, ssq = _pass1(
                True, nbr_cur, gp, base,
                p["emb_n"]["w"].T, p["emb_n"]["b"].reshape(1, 32),
                w1q, w2d, b2p)
        else:
            epa, epb, ssum, ssq = _pass1(
                False, nbr_cur, gp, base, None, None, w1q, w2d, b2p)
        s128 = jnp.sum(ssum, axis=0)
        q128 = jnp.sum(ssq, axis=0)
        s = (s128[:64] + s128[64:]).reshape(1, 64)
        q = (q128[:64] + q128[64:]).reshape(1, 64)
        mean = s / e_rows
        var = q / e_rows - mean * mean
        scale = bp["g1"].reshape(1, 64) / jnp.sqrt(var + 1e-5)
        shift = bp["bt1"].reshape(1, 64) - mean * scale
        scalep = jnp.concatenate([scale, scale], axis=1)
        shiftp = jnp.concatenate([shift, shift], axis=1)
        nbr_cur, em = _pass2(epa, epb, nbr_cur, scalep, shiftp)

        nargs = (bp["v1"]["w"].T, bp["v1"]["b"].reshape(1, 64),
                 bp["v2"]["w"].T, bp["v2"]["b"].reshape(1, 64),
                 bp["g2"].reshape(1, 64), bp["bt2"].reshape(1, 64),
                 bp["u"]["w"].T, bp["u"]["b"].reshape(1, 32),
                 bp["g3"].reshape(1, 32), bp["bt3"].reshape(1, 32))
        if k < 2:
            nx = e1s[k + 1]
            atom, state, pas, base = _node(
                em, atom, state, *nargs, nx[1], nx[2], nx[3], nx[4], nx[5])
        else:
            pooled = _node_last(em, atom, state, *nargs)

    bsz = surface_fea.shape[0]
    xs = surface_fea[:, 0, :, :]
    smem = pl.BlockSpec(memory_space=pltpu.SMEM)
    c1 = pl.pallas_call(
        _conv1_body,
        in_specs=[pl.BlockSpec(), smem, smem],
        out_shape=jax.ShapeDtypeStruct((3, bsz, 26, 26), F32),
    )(xs, p["c1w"], p["c1b"])
    c1d = c1[:, :, ::2, ::2]
    c2 = pl.pallas_call(
        _conv2_body,
        in_specs=[pl.BlockSpec(), smem, smem],
        out_shape=jax.ShapeDtypeStruct((6, bsz, 11, 11), F32),
    )(c1d, p["c2w"], p["c2b"])
    c2d = c2[:, :, ::2, ::2]
    sflat = jnp.transpose(c2d, (1, 0, 2, 3)).reshape(bsz, 216)

    out = pl.pallas_call(
        _head_body,
        out_shape=jax.ShapeDtypeStruct((bsz, 1), F32),
    )(pooled, sflat,
      p["f3"]["w"].T, p["f3"]["b"].reshape(1, 12),
      p["fc1"]["w"].T, p["fc1"]["b"].reshape(1, 128),
      p["fo"]["w"].T, p["fo"]["b"].reshape(1, 1))
    return out


# speedup vs baseline: 1.2040x; 1.0002x over previous
import functools

import jax
import jax.numpy as jnp
from jax import lax
from jax.experimental import pallas as pl
from jax.experimental.pallas import tpu as pltpu
from jax.experimental.pallas import tpu_sc as plsc

F32 = jnp.float32


def _dot(a, b):
    return jnp.dot(a, b, preferred_element_type=F32)


_SC_CHUNK = 128
_SC_NCHUNK = 40
_SC_PER_W = _SC_CHUNK * _SC_NCHUNK
_SC_NW = 32
_SC_EPAD = _SC_PER_W * _SC_NW
_SC_TOTCH = _SC_NCHUNK * _SC_NW


_SC_NBUF = 4
_SC_NCH0 = 56
_SC_NCH1 = 24


def _sc_gather(table, idx_pad):
    d = table.shape[1]
    dt = table.dtype
    mesh = plsc.VectorSubcoreMesh(core_axis_name="c", subcore_axis_name="s")

    @functools.partial(
        pl.kernel,
        out_type=jax.ShapeDtypeStruct((_SC_TOTCH, _SC_CHUNK, d), dt),
        mesh=mesh,
        scratch_types=(
            [pltpu.VMEM((_SC_NCH0 * _SC_CHUNK,), jnp.int32)]
            + [pltpu.VMEM((_SC_CHUNK, d), dt) for _ in range(_SC_NBUF)]
            + [pltpu.SemaphoreType.DMA for _ in range(2 * _SC_NBUF)]
        ),
        compiler_params=pltpu.CompilerParams(use_tc_tiling_on_sc=False),
    )
    def k(table_hbm, idx_hbm, out_hbm, idx_v, *bufsem):
        bufs = bufsem[:_SC_NBUF]
        gsems = bufsem[_SC_NBUF:2 * _SC_NBUF]
        wsems = bufsem[2 * _SC_NBUF:]
        cid = lax.axis_index("c")
        sid = lax.axis_index("s")

        def pipeline(nch, cbase):
            pltpu.sync_copy(
                idx_hbm.at[pl.ds(cbase * _SC_CHUNK, nch * _SC_CHUNK)],
                idx_v.at[pl.ds(0, nch * _SC_CHUNK)])

            def gather(c, b):
                return pltpu.async_copy(
                    table_hbm.at[idx_v.at[pl.ds(c * _SC_CHUNK, _SC_CHUNK)]],
                    bufs[b], gsems[b])

            for b in range(_SC_NBUF):
                gather(b, b)

            ngrp = nch // _SC_NBUF

            def grp_body(gi, _):
                c0 = gi * _SC_NBUF
                for b in range(_SC_NBUF):
                    pltpu.make_async_copy(
                        table_hbm.at[idx_v.at[pl.ds(0, _SC_CHUNK)]],
                        bufs[b], gsems[b]).wait()
                    pltpu.async_copy(bufs[b], out_hbm.at[cbase + c0 + b],
                                     wsems[b])
                for b in range(_SC_NBUF):
                    @pl.when(gi < ngrp - 1)
                    def _(b=b):
                        pltpu.make_async_copy(
                            bufs[b], out_hbm.at[cbase + c0 + b],
                            wsems[b]).wait()
                        gather(c0 + _SC_NBUF + b, b)
                return 0

            lax.fori_loop(0, ngrp, grp_body, 0, unroll=False)
            for b in range(_SC_NBUF):
                pltpu.make_async_copy(
                    bufs[b], out_hbm.at[cbase + (ngrp - 1) * _SC_NBUF + b],
                    wsems[b]).wait()

        @pl.when(cid == 0)
        def _():
            pipeline(_SC_NCH0, sid * _SC_NCH0)

        @pl.when(cid == 1)
        def _():
            pipeline(_SC_NCH1, 16 * _SC_NCH0 + sid * _SC_NCH1)

    return k(table, idx_pad)


def _prologue_body(af_ref, sf_ref, wat, ba, wst, bs, want, wsnt, waet, wset, be1,
                   atom_ref, state_ref, pas_ref, base_ref):
    atom = _dot(af_ref[...], wat[...]) + ba[...]
    state = _dot(sf_ref[...], wst[...]) + bs[...]
    atom_ref[...] = atom
    state_ref[...] = state
    pas_ref[...] = _dot(atom, want[...]) + _dot(state, wsnt[...])
    base_ref[...] = _dot(atom, waet[...]) + _dot(state, wset[...]) + be1[...]


def _prologue(atom_fea, state_fea, wat, ba, wst, bs, want, wsnt, waet, wset, be1):
    n = atom_fea.shape[0]
    return pl.pallas_call(
        _prologue_body,
        out_shape=(
            jax.ShapeDtypeStruct((n, 32), F32),
            jax.ShapeDtypeStruct((n, 32), F32),
            jax.ShapeDtypeStruct((n, 64), F32),
            jax.ShapeDtypeStruct((n, 64), F32),
        ),
    )(atom_fea, state_fea, wat, ba, wst, bs, want, wsnt, waet, wset, be1)


_NB = 400
_R4 = _NB * 4
_NGRID = 25
_E4 = _R4 * _NGRID


def _pass1_body(first, *args):
    if first:
        (raw_ref, gpa_ref, gpb_ref, base_ref, wnt_ref,
         bn_ref, w1q_ref, w2d_ref, b2p_ref,
         epa_ref, epb_ref, nbrp_out_ref, ssum_ref, ssq_ref) = args
        raw = raw_ref[...]
        nf = raw.shape[-1]
        embs = [_dot(raw[:, 4 * q:4 * q + 4, :].reshape(_R4, nf), wnt_ref[...])
                + bn_ref[...] for q in range(4)]
        nbrp = jnp.concatenate(embs, axis=1)
        nbrp_out_ref[...] = nbrp
    else:
        (nbr_ref, gpa_ref, gpb_ref, base_ref,
         w1q_ref, w2d_ref, b2p_ref,
         epa_ref, epb_ref, ssum_ref, ssq_ref) = args
        nbrp = nbr_ref[...]
    znbr4 = _dot(nbrp, w1q_ref[...])
    base = base_ref[...]
    br = jnp.broadcast_to(base[:, None, :], (_NB, 4, 64)).reshape(_R4, 64)
    basep = jnp.concatenate([br, br], axis=1)
    zpa = jnp.concatenate([znbr4[:, 0:64], znbr4[:, 128:192]], axis=1) \
        + gpa_ref[...] + basep
    zpb = jnp.concatenate([znbr4[:, 64:128], znbr4[:, 192:256]], axis=1) \
        + gpb_ref[...] + basep
    epa = _dot(jnp.tanh(zpa), w2d_ref[...]) + b2p_ref[...]
    epb = _dot(jnp.tanh(zpb), w2d_ref[...]) + b2p_ref[...]
    epa_ref[...] = epa
    epb_ref[...] = epb
    part = (jnp.sum(epa.reshape(_R4 // 8, 8, 128), axis=0)
            + jnp.sum(epb.reshape(_R4 // 8, 8, 128), axis=0))
    part2 = (jnp.sum((epa * epa).reshape(_R4 // 8, 8, 128), axis=0)
             + jnp.sum((epb * epb).reshape(_R4 // 8, 8, 128), axis=0))

    @pl.when(pl.program_id(0) == 0)
    def _():
        ssum_ref[...] = part
        ssq_ref[...] = part2

    @pl.when(pl.program_id(0) != 0)
    def _():
        ssum_ref[...] += part
        ssq_ref[...] += part2


def _pass1(first, nbr_in, gp, base, wnt, bn, w1q, w2d, b2p):
    full = lambda s: pl.BlockSpec(s, lambda i: (0, 0))
    out_shape = [jax.ShapeDtypeStruct((_E4, 128), F32)] * 2
    out_specs = [pl.BlockSpec((_R4, 128), lambda i: (i, 0))] * 2
    if first:
        out_shape.append(jax.ShapeDtypeStruct((_E4, 128), F32))
        out_specs.append(pl.BlockSpec((_R4, 128), lambda i: (i, 0)))
    out_shape += [jax.ShapeDtypeStruct((8, 128), F32)] * 2
    out_specs += [pl.BlockSpec((8, 128), lambda i: (0, 0))] * 2
    gp_specs = [pl.BlockSpec((_R4, 128), lambda i: (i, 0)),
                pl.BlockSpec((_R4, 128), lambda i: (i + _NGRID, 0))]
    base_spec = pl.BlockSpec((_NB, 64), lambda i: (i, 0))
    w_specs = [full((128, 256)), full((128, 128)), full((1, 128))]
    if first:
        nf = nbr_in.shape[-1]
        raw_spec = pl.BlockSpec((_NB, 16, nf), lambda i: (i, 0, 0))
        in_specs = ([raw_spec] + gp_specs + [base_spec]
                    + [full((nf, 32)), full((1, 32))] + w_specs)
        ins = (nbr_in, gp, gp, base, wnt, bn, w1q, w2d, b2p)
    else:
        in_specs = ([pl.BlockSpec((_R4, 128), lambda i: (i, 0))]
                    + gp_specs + [base_spec] + w_specs)
        ins = (nbr_in, gp, gp, base, w1q, w2d, b2p)
    return pl.pallas_call(
        functools.partial(_pass1_body, first),
        grid=(_NGRID,),
        in_specs=in_specs,
        out_specs=tuple(out_specs),
        out_shape=tuple(out_shape),
    )(*ins)


def _pass2_body(epa_ref, epb_ref, nbrp_ref, scale_ref, shift_ref,
                eop_ref, em_ref):
    eha = epa_ref[...] * scale_ref[...] + shift_ref[...]
    ehb = epb_ref[...] * scale_ref[...] + shift_ref[...]
    pa0 = eha[:, 0:32] * eha[:, 32:64]
    pa2 = eha[:, 64:96] * eha[:, 96:128]
    pb1 = ehb[:, 0:32] * ehb[:, 32:64]
    pb3 = ehb[:, 64:96] * ehb[:, 96:128]
    eop = jnp.concatenate([pa0, pb1, pa2, pb3], axis=1) + nbrp_ref[...]
    eop_ref[...] = eop
    s = jnp.sum(eop.reshape(_NB, 4, 128), axis=1)
    em_ref[...] = (s[:, 0:32] + s[:, 32:64] + s[:, 64:96] + s[:, 96:128]) \
        * (1.0 / 16.0)


def _pass2(epa, epb, nbrp, scalep, shiftp):
    n = _NB * _NGRID
    full = lambda s: pl.BlockSpec(s, lambda i: (0, 0))
    return pl.pallas_call(
        _pass2_body,
        grid=(_NGRID,),
        in_specs=[
            pl.BlockSpec((_R4, 128), lambda i: (i, 0)),
            pl.BlockSpec((_R4, 128), lambda i: (i, 0)),
            pl.BlockSpec((_R4, 128), lambda i: (i, 0)),
            full((1, 128)),
            full((1, 128)),
        ],
        out_specs=(pl.BlockSpec((_R4, 128), lambda i: (i, 0)),
                   pl.BlockSpec((_NB, 32), lambda i: (i, 0))),
        out_shape=(jax.ShapeDtypeStruct((_E4, 128), F32),
                   jax.ShapeDtypeStruct((n, 32), F32)),
    )(epa, epb, nbrp, scalep, shiftp)


def _bn_cols(x, g, b):
    n = x.shape[0]
    m = jnp.sum(x, axis=0, keepdims=True) / n
    var = jnp.sum(x * x, axis=0, keepdims=True) / n - m * m
    return (x - m) * (g / jnp.sqrt(var + 1e-5)) + b


def _node_body(em_ref, atom_ref, state_ref, v1t, bv1, v2t, bv2,
               g2, bt2, ut, bu, g3, bt3, want, wsnt, waet, wset, be1n,
               atom_o, state_o, pas_o, base_o):
    x = jnp.concatenate([em_ref[...], atom_ref[...]], axis=1)
    h = jnp.tanh(_dot(x, v1t[...]) + bv1[...])
    v = jnp.tanh(_dot(h, v2t[...]) + bv2[...])
    vn = _bn_cols(v, g2[...], bt2[...])
    atom_new = vn[:, :32] * vn[:, 32:] + atom_ref[...]
    su = _dot(state_ref[...], ut[...]) + bu[...]
    state_new = _bn_cols(su, g3[...], bt3[...]) + state_ref[...]
    atom_o[...] = atom_new
    state_o[...] = state_new
    pas_o[...] = _dot(atom_new, want[...]) + _dot(state_new, wsnt[...])
    base_o[...] = _dot(atom_new, waet[...]) + _dot(state_new, wset[...]) + be1n[...]


def _node(em, atom, state, v1t, bv1, v2t, bv2, g2, bt2, ut, bu, g3, bt3,
          want, wsnt, waet, wset, be1n):
    n = atom.shape[0]
    return pl.pallas_call(
        _node_body,
        out_shape=(
            jax.ShapeDtypeStruct((n, 32), F32),
            jax.ShapeDtypeStruct((n, 32), F32),
            jax.ShapeDtypeStruct((n, 64), F32),
            jax.ShapeDtypeStruct((n, 64), F32),
        ),
    )(em, atom, state, v1t, bv1, v2t, bv2, g2, bt2, ut, bu, g3, bt3,
      want, wsnt, waet, wset, be1n)


def _node_last_body(em_ref, atom_ref, state_ref, v1t, bv1, v2t,
                    bv2, g2, bt2, ut, bu, g3, bt3, pooled_o):
    em = em_ref[...]
    x = jnp.concatenate([em, atom_ref[...]], axis=1)
    h = jnp.tanh(_dot(x, v1t[...]) + bv1[...])
    v = jnp.tanh(_dot(h, v2t[...]) + bv2[...])
    vn = _bn_cols(v, g2[...], bt2[...])
    atom_new = vn[:, :32] * vn[:, 32:] + atom_ref[...]
    su = _dot(state_ref[...], ut[...]) + bu[...]
    state_new = _bn_cols(su, g3[...], bt3[...]) + state_ref[...]
    crys = jnp.concatenate([em * 16.0, atom_new, state_new], axis=1)
    n = crys.shape[0]
    pooled_o[...] = jnp.mean(crys.reshape(n // 50, 50, 96), axis=1)


def _node_last(em, atom, state, v1t, bv1, v2t, bv2, g2, bt2, ut, bu, g3, bt3):
    n = atom.shape[0]
    return pl.pallas_call(
        _node_last_body,
        out_shape=jax.ShapeDtypeStruct((n // 50, 96), F32),
    )(em, atom, state, v1t, bv1, v2t, bv2, g2, bt2, ut, bu, g3, bt3)


def _conv1_body(x_ref, w_ref, b_ref, o_ref):
    x = x_ref[...]
    bsz = x.shape[0]
    outs = []
    for o in range(3):
        acc = jnp.full((bsz, 27, 27), b_ref[o], dtype=F32)
        for dh in range(2):
            for dw in range(2):
                acc = acc + w_ref[o, 0, dh, dw] * x[:, dh:dh + 27, dw:dw + 27]
        m = jnp.maximum(
            jnp.maximum(acc[:, 0:26, 0:26], acc[:, 0:26, 1:27]),
            jnp.maximum(acc[:, 1:27, 0:26], acc[:, 1:27, 1:27]),
        )
        outs.append(jnp.maximum(m, 0.0))
    o_ref[...] = jnp.stack(outs, axis=0)


def _conv2_body(x_ref, w_ref, b_ref, o_ref):
    x = x_ref[...]
    bsz = x.shape[1]
    outs = []
    for o in range(6):
        acc = jnp.full((bsz, 12, 12), b_ref[o], dtype=F32)
        for ci in range(3):
            for dh in range(2):
                for dw in range(2):
                    acc = acc + w_ref[o, ci, dh, dw] * x[ci, :, dh:dh + 12, dw:dw + 12]
        m = jnp.maximum(
            jnp.maximum(acc[:, 0:11, 0:11], acc[:, 0:11, 1:12]),
            jnp.maximum(acc[:, 1:12, 0:11], acc[:, 1:12, 1:12]),
        )
        outs.append(jnp.maximum(m, 0.0))
    o_ref[...] = jnp.stack(outs, axis=0)


def _head_body(pooled_ref, sflat_ref, wf3t, bf3, wfc1t, bfc1, wfot, bfo, o_ref):
    s12 = jnp.tanh(_dot(sflat_ref[...], wf3t[...]) + bf3[...])
    x = jnp.concatenate([pooled_ref[...], s12], axis=1)
    h = jnp.tanh(_dot(x, wfc1t[...]) + bfc1[...])
    o_ref[...] = _dot(h, wfot[...]) + bfo[...]


def _split_e1(p):
    w = p["e1"]["w"]
    return (w[:, 64:96].T, w[:, 96:128].T, w[:, 128:160].T,
            w[:, 0:32].T, w[:, 32:64].T, p["e1"]["b"].reshape(1, 64))


def _blockdiag(wt, copies):
    i, o = wt.shape
    z = jnp.zeros((copies, i, copies, o), F32)
    ii = jnp.arange(copies)
    z = z.at[ii, :, ii, :].set(wt)
    return z.reshape(copies * i, copies * o)


def kernel(atom_fea, nbr_fea, nbr_fea_idx, state_fea, surface_fea,
           crystal_atom_idx, params):
    p = params
    blocks = p["blocks"]
    n, m = nbr_fea_idx.shape
    e_rows = n * m

    e1s = [_split_e1(bp) for bp in blocks]

    atom, state, pas, base = _prologue(
        atom_fea, state_fea,
        p["emb_a"]["w"].T, p["emb_a"]["b"].reshape(1, 32),
        p["emb_s"]["w"].T, p["emb_s"]["b"].reshape(1, 32),
        e1s[0][1], e1s[0][2], e1s[0][3], e1s[0][4], e1s[0][5])

    idx_flat = nbr_fea_idx.reshape(e_rows)
    pp = jnp.arange(_SC_EPAD, dtype=jnp.int32)
    half = jnp.where(pp < e_rows // 2, 0, 1)
    q = pp - half * (e_rows // 2)
    n_ = q >> 3
    jr = (q >> 1) & 3
    h = q & 1
    src = n_ * 16 + half * 4 + jr + h * 8
    src = jnp.where(pp < e_rows, src, 0)
    idx_pad = jnp.take(idx_flat, src)

    nbr_cur = nbr_fea
    for k in range(3):
        bp = blocks[k]
        wnbr_t, _, _, _, _, be1 = e1s[k]
        w1q = _blockdiag(wnbr_t, 4)
        w2d = _blockdiag(bp["e2"]["w"].T, 2)
        b2 = bp["e2"]["b"].reshape(1, 64)
        b2p = jnp.concatenate([b2, b2], axis=1)
        g3d = _sc_gather(pas, idx_pad)
        gp = g3d.reshape(_SC_EPAD * 64 // 128, 128)
        if k == 0:
            epa, epb, nbr_cur, ssum, ssq = _pass1(
                True, nbr_cur, gp, base,
                p["emb_n"]["w"].T, p["emb_n"]["b"].reshape(1, 32),
                w1q, w2d, b2p)
        else:
            epa, epb, ssum, ssq = _pass1(
                False, nbr_cur, gp, base, None, None, w1q, w2d, b2p)
        s128 = jnp.sum(ssum, axis=0)
        q128 = jnp.sum(ssq, axis=0)
        s = (s128[:64] + s128[64:]).reshape(1, 64)
        q = (q128[:64] + q128[64:]).reshape(1, 64)
        mean = s / e_rows
        var = q / e_rows - mean * mean
        scale = bp["g1"].reshape(1, 64) / jnp.sqrt(var + 1e-5)
        shift = bp["bt1"].reshape(1, 64) - mean * scale
        scalep = jnp.concatenate([scale, scale], axis=1)
        shiftp = jnp.concatenate([shift, shift], axis=1)
        nbr_cur, em = _pass2(epa, epb, nbr_cur, scalep, shiftp)

        nargs = (bp["v1"]["w"].T, bp["v1"]["b"].reshape(1, 64),
                 bp["v2"]["w"].T, bp["v2"]["b"].reshape(1, 64),
                 bp["g2"].reshape(1, 64), bp["bt2"].reshape(1, 64),
                 bp["u"]["w"].T, bp["u"]["b"].reshape(1, 32),
                 bp["g3"].reshape(1, 32), bp["bt3"].reshape(1, 32))
        if k < 2:
            nx = e1s[k + 1]
            atom, state, pas, base = _node(
                em, atom, state, *nargs, nx[1], nx[2], nx[3], nx[4], nx[5])
        else:
            pooled = _node_last(em, atom, state, *nargs)

    bsz = surface_fea.shape[0]
    xs = surface_fea[:, 0, :, :]
    smem = pl.BlockSpec(memory_space=pltpu.SMEM)
    c1 = pl.pallas_call(
        _conv1_body,
        in_specs=[pl.BlockSpec(), smem, smem],
        out_shape=jax.ShapeDtypeStruct((3, bsz, 26, 26), F32),
    )(xs, p["c1w"], p["c1b"])
    c1d = c1[:, :, ::2, ::2]
    c2 = pl.pallas_call(
        _conv2_body,
        in_specs=[pl.BlockSpec(), smem, smem],
        out_shape=jax.ShapeDtypeStruct((6, bsz, 11, 11), F32),
    )(c1d, p["c2w"], p["c2b"])
    c2d = c2[:, :, ::2, ::2]
    sflat = jnp.transpose(c2d, (1, 0, 2, 3)).reshape(bsz, 216)

    out = pl.pallas_call(
        _head_body,
        out_shape=jax.ShapeDtypeStruct((bsz, 1), F32),
    )(pooled, sflat,
      p["f3"]["w"].T, p["f3"]["b"].reshape(1, 12),
      p["fc1"]["w"].T, p["fc1"]["b"].reshape(1, 128),
      p["fo"]["w"].T, p["fo"]["b"].reshape(1, 1))
    return out
